# Initial kernel scaffold; baseline (speedup 1.0000x reference)
import jax
import jax.numpy as jnp
from jax.experimental import pallas as pl


def kernel(x, edge_index, W1, b1, eps1, W2, b2, eps2, W3, b3, eps3):
    raise NotImplementedError("write your pallas kernel here")


# speedup vs baseline: 2.9647x; 2.9647x over previous
import functools

import jax
import jax.numpy as jnp
from jax import lax
from jax.experimental import pallas as pl
from jax.experimental.pallas import tpu as pltpu
from jax.experimental.pallas import tpu_sc as plsc

N = 10000
E = 320000
D = 128

NC = 2
NS = 16
NW = NC * NS
CHUNK = 128
CPT = 80
SLAB = 64
EPW = CHUNK * CPT
E_PAD = NW * EPW
ACC_ROWS = 10112
ZPT = ACC_ROWS // NS

_mesh = plsc.VectorSubcoreMesh(core_axis_name="c", subcore_axis_name="s")


@functools.partial(
    pl.kernel,
    out_type=jax.ShapeDtypeStruct((NC, ACC_ROWS, D), jnp.float32),
    mesh=_mesh,
    scratch_types=[
        pltpu.VMEM((SLAB, CHUNK), jnp.int32),
        pltpu.VMEM((SLAB, CHUNK), jnp.int32),
        pltpu.VMEM((CHUNK, D), jnp.float32),
        pltpu.VMEM((CHUNK, D), jnp.float32),
        pltpu.VMEM_SHARED((ACC_ROWS, D), jnp.float32),
        pltpu.SemaphoreType.DMA,
        pltpu.SemaphoreType.DMA,
    ],
)
def _sc_agg(h_hbm, src_hbm, dst_hbm, zeros_hbm, out_hbm,
            src_v, dst_v, buf0, buf1, acc, sem0, sem1):
    cid = lax.axis_index("c")
    sid = lax.axis_index("s")
    wid = sid * NC + cid

    pltpu.sync_copy(zeros_hbm.at[pl.ds(sid * ZPT, ZPT)],
                    acc.at[pl.ds(sid * ZPT, ZPT)])
    plsc.subcore_barrier()

    def process_section(base, nc):
        pltpu.sync_copy(src_hbm.at[wid].at[pl.ds(base, nc)],
                        src_v.at[pl.ds(0, nc)])
        pltpu.sync_copy(dst_hbm.at[wid].at[pl.ds(base, nc)],
                        dst_v.at[pl.ds(0, nc)])
        pltpu.async_copy(h_hbm.at[src_v.at[0]], buf0, sem0)

        @pl.loop(0, nc, step=2)
        def _(j):
            pltpu.async_copy(h_hbm.at[src_v.at[j + 1]], buf1, sem1)
            pltpu.make_async_copy(h_hbm.at[src_v.at[j]], buf0, sem0).wait()
            pltpu.sync_copy(buf0, acc.at[dst_v.at[j]], add=True)

            @pl.when(j + 2 < nc)
            def _():
                pltpu.async_copy(h_hbm.at[src_v.at[j + 2]], buf0, sem0)

            pltpu.make_async_copy(h_hbm.at[src_v.at[j + 1]], buf1, sem1).wait()
            pltpu.sync_copy(buf1, acc.at[dst_v.at[j + 1]], add=True)

    process_section(0, SLAB)
    process_section(SLAB, CPT - SLAB)

    plsc.subcore_barrier()
    pltpu.sync_copy(acc.at[pl.ds(sid * ZPT, ZPT)],
                    out_hbm.at[cid].at[pl.ds(sid * ZPT, ZPT)])


def _tc_body(h_ref, p_ref, w_ref, b_ref, s_ref, o_ref, *, relu):
    z = s_ref[0, 0] * h_ref[...] + (p_ref[0] + p_ref[1])
    y = lax.dot_general(z, w_ref[...], (((1,), (1,)), ((), ())),
                        preferred_element_type=jnp.float32)
    y = y + b_ref[...]
    o_ref[...] = jnp.maximum(y, 0.0) if relu else y


def _tc_layer(relu):
    return pl.pallas_call(
        functools.partial(_tc_body, relu=relu),
        out_shape=jax.ShapeDtypeStruct((N, D), jnp.float32),
    )


def kernel(x, edge_index, W1, b1, eps1, W2, b2, eps2, W3, b3, eps3):
    src = edge_index[0].astype(jnp.int32)
    dst = edge_index[1].astype(jnp.int32)
    pad = E_PAD - E
    src_p = jnp.concatenate([src, jnp.zeros((pad,), jnp.int32)]).reshape(
        NW, CPT, CHUNK)
    dst_p = jnp.concatenate([dst, jnp.full((pad,), N, jnp.int32)]).reshape(
        NW, CPT, CHUNK)
    zeros = jnp.zeros((ACC_ROWS, D), jnp.float32)

    h = x
    for W, b, eps, relu in ((W1, b1, eps1, True),
                            (W2, b2, eps2, True),
                            (W3, b3, eps3, False)):
        parts = _sc_agg(h, src_p, dst_p, zeros)[:, :N]
        scale = jnp.reshape(1.0 + eps, (1, 1)).astype(jnp.float32)
        h = _tc_layer(relu)(h, parts, W, b.reshape(1, D), scale)
    return h
